# Initial kernel scaffold; baseline (speedup 1.0000x reference)
import jax
import jax.numpy as jnp
import numpy as np
from jax.experimental import pallas as pl

_B, _P, _DIN, _H, _K = 256, 256, 4, 64, 16
_BIG = jnp.float32(3e38)


def _edge(h, P, W1d, W1b, b1, W2, b2, Eev, Eod):
    f32 = jnp.float32
    hn = jnp.dot(h, W1b, preferred_element_type=f32)
    hc = jnp.dot(h, W1d, preferred_element_type=f32) + b1
    sq = jnp.sum(h * h, axis=1, keepdims=True)
    hhT = jax.lax.dot_general(h, h, (((1,), (1,)), ((), ())),
                              preferred_element_type=f32)
    D = sq - 2.0 * hhT
    fiota = jax.lax.broadcasted_iota(f32, (P, P), 0)
    S = jnp.zeros((P, _H), f32)
    for _ in range(_K):
        colmin = jnp.min(D, axis=0, keepdims=True)
        cand = jnp.where(D == colmin, fiota, f32(P))
        minidx = jnp.min(cand, axis=0, keepdims=True)
        sel = fiota == minidx
        onehot = sel.astype(f32)
        G = jax.lax.dot_general(onehot, hn, (((0,), (0,)), ((), ())),
                                preferred_element_type=f32)
        S = S + jnp.maximum(hc + G, 0.0)
        D = jnp.where(sel, _BIG, D)
    agg = jnp.dot(S, W2, preferred_element_type=f32) + f32(_K) * b2
    ev = jnp.dot(Eev, agg, preferred_element_type=f32)
    od = jnp.dot(Eod, agg, preferred_element_type=f32)
    return jnp.maximum(ev, od)


def _graph_kernel(x_ref, W_in_ref, b_in_ref,
                  W1d0_ref, W1b0_ref, b10_ref, W20_ref, b20_ref,
                  W1d1_ref, W1b1_ref, b11_ref, W21_ref, b21_ref,
                  Eev0_ref, Eod0_ref, Eev1_ref, Eod1_ref,
                  g_ref):
    f32 = jnp.float32
    x = x_ref[0]
    h = jnp.maximum(
        jnp.dot(x, W_in_ref[...], preferred_element_type=f32) + b_in_ref[...],
        0.0)
    h = _edge(h, _P, W1d0_ref[...], W1b0_ref[...], b10_ref[...],
              W20_ref[...], b20_ref[...], Eev0_ref[...], Eod0_ref[...])
    h = _edge(h, _P // 2, W1d1_ref[...], W1b1_ref[...], b11_ref[...],
              W21_ref[...], b21_ref[...], Eev1_ref[...], Eod1_ref[...])
    g_ref[...] = jnp.max(h, axis=0, keepdims=True)


def _out_kernel(g_ref, gx_ref, Wg_ref, Wx_ref, bo1_ref, Wo2_ref, bo2_ref,
                Wo3_ref, bo3_ref, o_ref):
    f32 = jnp.float32
    t = (jnp.dot(g_ref[...], Wg_ref[...], preferred_element_type=f32)
         + jnp.dot(gx_ref[...], Wx_ref[...], preferred_element_type=f32)
         + bo1_ref[...])
    t = jnp.maximum(t, 0.0)
    t = jnp.maximum(
        jnp.dot(t, Wo2_ref[...], preferred_element_type=f32) + bo2_ref[...],
        0.0)
    o_ref[...] = (jnp.dot(t, Wo3_ref[...], preferred_element_type=f32)
                  + bo3_ref[...])


def _pair_selectors(P):
    ev = np.zeros((P // 2, P), np.float32)
    od = np.zeros((P // 2, P), np.float32)
    j = np.arange(P // 2)
    ev[j, 2 * j] = 1.0
    od[j, 2 * j + 1] = 1.0
    return jnp.asarray(ev), jnp.asarray(od)


def kernel(x, gx, W_in, b_in, W1_0, b1_0, W2_0, b2_0, W1_1, b1_1, W2_1,
           b2_1, Wo1, bo1, Wo2, bo2, Wo3, bo3):
    f32 = jnp.float32
    W1d0, W1b0 = W1_0[:_H] - W1_0[_H:], W1_0[_H:]
    W1d1, W1b1 = W1_1[:_H] - W1_1[_H:], W1_1[_H:]
    Eev0, Eod0 = _pair_selectors(_P)
    Eev1, Eod1 = _pair_selectors(_P // 2)
    row = lambda v: v.reshape(1, -1).astype(f32)

    full = lambda a: pl.BlockSpec(a.shape, (lambda nd: lambda b: (0,) * nd)(a.ndim))
    consts = (W_in, row(b_in), W1d0, W1b0, row(b1_0), W2_0, row(b2_0),
              W1d1, W1b1, row(b1_1), W2_1, row(b2_1),
              Eev0, Eod0, Eev1, Eod1)
    g = pl.pallas_call(
        _graph_kernel,
        grid=(_B,),
        in_specs=[pl.BlockSpec((1, _P, _DIN), lambda b: (b, 0, 0))]
        + [full(a) for a in consts],
        out_specs=pl.BlockSpec((1, _H), lambda b: (b, 0)),
        out_shape=jax.ShapeDtypeStruct((_B, _H), f32),
    )(x, *consts)

    oconsts = (Wo1[:_H], Wo1[_H:], row(bo1), Wo2, row(bo2), Wo3, row(bo3))
    out = pl.pallas_call(
        _out_kernel,
        in_specs=[pl.BlockSpec((_B, _H), lambda: (0, 0)),
                  pl.BlockSpec((_B, _DIN), lambda: (0, 0))]
        + [pl.BlockSpec(a.shape, (lambda nd: lambda: (0,) * nd)(a.ndim))
           for a in oconsts],
        out_specs=pl.BlockSpec((_B, 1), lambda: (0, 0)),
        out_shape=jax.ShapeDtypeStruct((_B, 1), f32),
    )(g, gx, *oconsts)
    return out


# speedup vs baseline: 10.8778x; 10.8778x over previous
import jax
import jax.numpy as jnp
import numpy as np
from jax.experimental import pallas as pl

_B, _P, _DIN, _H, _K = 256, 256, 4, 64, 16
_BIG = np.float32(3e38)
_HIGH = jax.lax.Precision.HIGHEST
_NT = (((1,), (1,)), ((), ()))
_TN = (((0,), (0,)), ((), ()))


def _split3(h):
    bf = jnp.bfloat16
    hi = h.astype(bf)
    r1 = h - hi.astype(jnp.float32)
    mid = r1.astype(bf)
    lo = (r1 - mid.astype(jnp.float32)).astype(bf)
    return hi, mid, lo


def _edge(h, P, W1a, W1b, b1, W2, b2, Eev, Eod):
    f32, bf = np.float32, jnp.bfloat16
    hb = h.astype(bf)
    hh = jax.lax.dot_general(hb, hb, _NT, preferred_element_type=f32)
    sq = jnp.sum(h * h, axis=1, keepdims=True)
    eye = (jax.lax.broadcasted_iota(jnp.int32, (P, P), 0)
           == jax.lax.broadcasted_iota(jnp.int32, (P, P), 1)).astype(f32)
    sq_row = jax.lax.dot_general(sq, eye, _TN, preferred_element_type=f32,
                                 precision=_HIGH)
    D = (sq + sq_row) - 2.0 * hh
    t1 = jax.lax.dot_general(hb, W1a, (((1,), (0,)), ((), ())),
                             preferred_element_type=f32)
    g_hi, g_mid, g_lo = _split3(h)
    fiota = jax.lax.broadcasted_iota(jnp.int32, (P, P), 0).astype(f32)
    S = jnp.zeros((P, _H), f32)
    for _ in range(_K):
        colmin = jnp.min(D, axis=0, keepdims=True)
        cand = jnp.where(D == colmin, fiota, f32(P))
        minidx = jnp.min(cand, axis=0, keepdims=True)
        sel = fiota == minidx
        oh = sel.astype(bf)
        G = (jax.lax.dot_general(oh, g_hi, _TN, preferred_element_type=f32)
             + jax.lax.dot_general(oh, g_mid, _TN, preferred_element_type=f32)
             + jax.lax.dot_general(oh, g_lo, _TN, preferred_element_type=f32))
        dk = (G - h).astype(bf)
        m = jnp.maximum(
            (t1 + jnp.dot(dk, W1b, preferred_element_type=f32)) + b1, 0.0)
        S = S + jnp.dot(m.astype(bf), W2, preferred_element_type=f32)
        D = jnp.where(sel, _BIG, D)
    agg = S + f32(_K) * b2
    ev = jnp.dot(Eev, agg, preferred_element_type=f32, precision=_HIGH)
    od = jnp.dot(Eod, agg, preferred_element_type=f32, precision=_HIGH)
    return jnp.maximum(ev, od)


def _graph_kernel(x_ref, W_in_ref, b_in_ref,
                  W1a0_ref, W1b0_ref, b10_ref, W20_ref, b20_ref,
                  W1a1_ref, W1b1_ref, b11_ref, W21_ref, b21_ref,
                  Eev0_ref, Eod0_ref, Eev1_ref, Eod1_ref,
                  g_ref):
    f32 = np.float32
    x = x_ref[0].astype(jnp.bfloat16)
    h = jnp.maximum(
        jnp.dot(x, W_in_ref[...], preferred_element_type=f32) + b_in_ref[...],
        0.0)
    h = _edge(h, _P, W1a0_ref[...], W1b0_ref[...], b10_ref[...],
              W20_ref[...], b20_ref[...], Eev0_ref[...], Eod0_ref[...])
    h = _edge(h, _P // 2, W1a1_ref[...], W1b1_ref[...], b11_ref[...],
              W21_ref[...], b21_ref[...], Eev1_ref[...], Eod1_ref[...])
    g_ref[...] = jnp.max(h, axis=0, keepdims=True)[None]


def _out_kernel(g_ref, gx_ref, Wg_ref, Wx_ref, bo1_ref, Wo2_ref, bo2_ref,
                Wo3_ref, bo3_ref, o_ref):
    f32, bf = np.float32, jnp.bfloat16
    t = (jnp.dot(g_ref[...].astype(bf), Wg_ref[...], preferred_element_type=f32)
         + jnp.dot(gx_ref[...].astype(bf), Wx_ref[...], preferred_element_type=f32)
         + bo1_ref[...])
    t = jnp.maximum(t, 0.0)
    t = jnp.maximum(
        jnp.dot(t.astype(bf), Wo2_ref[...], preferred_element_type=f32)
        + bo2_ref[...], 0.0)
    o_ref[...] = (jnp.dot(t.astype(bf), Wo3_ref[...], preferred_element_type=f32)
                  + bo3_ref[...])


def _pair_selectors(P):
    ev = np.zeros((P // 2, P), np.float32)
    od = np.zeros((P // 2, P), np.float32)
    j = np.arange(P // 2)
    ev[j, 2 * j] = 1.0
    od[j, 2 * j + 1] = 1.0
    return jnp.asarray(ev), jnp.asarray(od)


def kernel(x, gx, W_in, b_in, W1_0, b1_0, W2_0, b2_0, W1_1, b1_1, W2_1,
           b2_1, Wo1, bo1, Wo2, bo2, Wo3, bo3):
    f32, bf = np.float32, jnp.bfloat16
    Eev0, Eod0 = _pair_selectors(_P)
    Eev1, Eod1 = _pair_selectors(_P // 2)
    row = lambda v: v.reshape(1, -1).astype(f32)

    full = lambda a: pl.BlockSpec(a.shape, (lambda nd: lambda b: (0,) * nd)(a.ndim))
    consts = (W_in.astype(bf), row(b_in),
              W1_0[:_H].astype(bf), W1_0[_H:].astype(bf), row(b1_0),
              W2_0.astype(bf), row(b2_0),
              W1_1[:_H].astype(bf), W1_1[_H:].astype(bf), row(b1_1),
              W2_1.astype(bf), row(b2_1),
              Eev0, Eod0, Eev1, Eod1)
    g = pl.pallas_call(
        _graph_kernel,
        grid=(_B,),
        in_specs=[pl.BlockSpec((1, _P, _DIN), lambda b: (b, 0, 0))]
        + [full(a) for a in consts],
        out_specs=pl.BlockSpec((1, 1, _H), lambda b: (b, 0, 0)),
        out_shape=jax.ShapeDtypeStruct((_B, 1, _H), f32),
    )(x, *consts)
    g = g.reshape(_B, _H)

    oconsts = (Wo1[:_H].astype(bf), Wo1[_H:].astype(bf), row(bo1),
               Wo2.astype(bf), row(bo2), Wo3.astype(bf), row(bo3))
    out = pl.pallas_call(
        _out_kernel,
        in_specs=[pl.BlockSpec((_B, _H), lambda: (0, 0)),
                  pl.BlockSpec((_B, _DIN), lambda: (0, 0))]
        + [pl.BlockSpec(a.shape, (lambda nd: lambda: (0,) * nd)(a.ndim))
           for a in oconsts],
        out_specs=pl.BlockSpec((_B, 1), lambda: (0, 0)),
        out_shape=jax.ShapeDtypeStruct((_B, 1), f32),
    )(g, gx, *oconsts)
    return out


# speedup vs baseline: 12.3379x; 1.1342x over previous
import jax
import jax.numpy as jnp
import numpy as np
from jax.experimental import pallas as pl

_B, _P, _DIN, _H, _K = 256, 256, 4, 64, 16
_G = 4
_BIG = np.float32(3e38)
_HIGH = jax.lax.Precision.HIGHEST
_NT = (((1,), (1,)), ((), ()))
_TN = (((0,), (0,)), ((), ()))


def _split3(h):
    bf = jnp.bfloat16
    hi = h.astype(bf)
    r1 = h - hi.astype(jnp.float32)
    mid = r1.astype(bf)
    lo = (r1 - mid.astype(jnp.float32)).astype(bf)
    return hi, mid, lo


def _edge_multi(hs, P, W1a, W1b, b1, W2, b2):
    f32, bf = np.float32, jnp.bfloat16
    eye = (jax.lax.broadcasted_iota(jnp.int32, (P, P), 0)
           == jax.lax.broadcasted_iota(jnp.int32, (P, P), 1)).astype(f32)
    fiota = jax.lax.broadcasted_iota(jnp.int32, (P, P), 0).astype(f32)
    st = []
    for h in hs:
        hb = h.astype(bf)
        hh = jax.lax.dot_general(hb, hb, _NT, preferred_element_type=f32)
        sq = jnp.sum(h * h, axis=1, keepdims=True)
        sq_row = jax.lax.dot_general(sq, eye, _TN, preferred_element_type=f32,
                                     precision=_HIGH)
        D = (sq + sq_row) - 2.0 * hh
        t1 = jax.lax.dot_general(hb, W1a, (((1,), (0,)), ((), ())),
                                 preferred_element_type=f32)
        st.append({"h": h, "D": D, "t1": t1, "g3": _split3(h),
                   "S": jnp.zeros((P, _H), f32)})
    for _ in range(_K):
        for s in st:
            colmin = jnp.min(s["D"], axis=0, keepdims=True)
            cand = jnp.where(s["D"] == colmin, fiota, f32(P))
            minidx = jnp.min(cand, axis=0, keepdims=True)
            s["sel"] = fiota == minidx
        for s in st:
            g_hi, g_mid, g_lo = s["g3"]
            oh = s["sel"].astype(bf)
            G = (jax.lax.dot_general(oh, g_hi, _TN, preferred_element_type=f32)
                 + jax.lax.dot_general(oh, g_mid, _TN, preferred_element_type=f32)
                 + jax.lax.dot_general(oh, g_lo, _TN, preferred_element_type=f32))
            dk = (G - s["h"]).astype(bf)
            m = jnp.maximum(
                (s["t1"] + jnp.dot(dk, W1b, preferred_element_type=f32)) + b1,
                0.0)
            s["S"] = s["S"] + jnp.dot(m.astype(bf), W2,
                                      preferred_element_type=f32)
            s["D"] = jnp.where(s["sel"], _BIG, s["D"])
    outs = []
    for s in st:
        agg = s["S"] + f32(_K) * b2
        outs.append(jnp.max(agg.reshape(P // 2, 2, _H), axis=1))
    return outs


def _graph_kernel(x_ref, W_in_ref, b_in_ref,
                  W1a0_ref, W1b0_ref, b10_ref, W20_ref, b20_ref,
                  W1a1_ref, W1b1_ref, b11_ref, W21_ref, b21_ref,
                  g_ref):
    f32 = np.float32
    hs = []
    for i in range(_G):
        x = x_ref[i].astype(jnp.bfloat16)
        hs.append(jnp.maximum(
            jnp.dot(x, W_in_ref[...], preferred_element_type=f32)
            + b_in_ref[...], 0.0))
    hs = _edge_multi(hs, _P, W1a0_ref[...], W1b0_ref[...], b10_ref[...],
                     W20_ref[...], b20_ref[...])
    hs = _edge_multi(hs, _P // 2, W1a1_ref[...], W1b1_ref[...], b11_ref[...],
                     W21_ref[...], b21_ref[...])
    for i in range(_G):
        g_ref[i] = jnp.max(hs[i], axis=0, keepdims=True)


def _out_kernel(g_ref, gx_ref, Wg_ref, Wx_ref, bo1_ref, Wo2_ref, bo2_ref,
                Wo3_ref, bo3_ref, o_ref):
    f32, bf = np.float32, jnp.bfloat16
    t = (jnp.dot(g_ref[...].astype(bf), Wg_ref[...], preferred_element_type=f32)
         + jnp.dot(gx_ref[...].astype(bf), Wx_ref[...], preferred_element_type=f32)
         + bo1_ref[...])
    t = jnp.maximum(t, 0.0)
    t = jnp.maximum(
        jnp.dot(t.astype(bf), Wo2_ref[...], preferred_element_type=f32)
        + bo2_ref[...], 0.0)
    o_ref[...] = (jnp.dot(t.astype(bf), Wo3_ref[...], preferred_element_type=f32)
                  + bo3_ref[...])


def kernel(x, gx, W_in, b_in, W1_0, b1_0, W2_0, b2_0, W1_1, b1_1, W2_1,
           b2_1, Wo1, bo1, Wo2, bo2, Wo3, bo3):
    f32, bf = np.float32, jnp.bfloat16
    row = lambda v: v.reshape(1, -1).astype(f32)

    full = lambda a: pl.BlockSpec(a.shape, (lambda nd: lambda b: (0,) * nd)(a.ndim))
    consts = (W_in.astype(bf), row(b_in),
              W1_0[:_H].astype(bf), W1_0[_H:].astype(bf), row(b1_0),
              W2_0.astype(bf), row(b2_0),
              W1_1[:_H].astype(bf), W1_1[_H:].astype(bf), row(b1_1),
              W2_1.astype(bf), row(b2_1))
    g = pl.pallas_call(
        _graph_kernel,
        grid=(_B // _G,),
        in_specs=[pl.BlockSpec((_G, _P, _DIN), lambda b: (b, 0, 0))]
        + [full(a) for a in consts],
        out_specs=pl.BlockSpec((_G, 1, _H), lambda b: (b, 0, 0)),
        out_shape=jax.ShapeDtypeStruct((_B, 1, _H), f32),
    )(x, *consts)
    g = g.reshape(_B, _H)

    oconsts = (Wo1[:_H].astype(bf), Wo1[_H:].astype(bf), row(bo1),
               Wo2.astype(bf), row(bo2), Wo3.astype(bf), row(bo3))
    out = pl.pallas_call(
        _out_kernel,
        in_specs=[pl.BlockSpec((_B, _H), lambda: (0, 0)),
                  pl.BlockSpec((_B, _DIN), lambda: (0, 0))]
        + [pl.BlockSpec(a.shape, (lambda nd: lambda: (0,) * nd)(a.ndim))
           for a in oconsts],
        out_specs=pl.BlockSpec((_B, 1), lambda: (0, 0)),
        out_shape=jax.ShapeDtypeStruct((_B, 1), f32),
    )(g, gx, *oconsts)
    return out
